# Initial kernel scaffold; baseline (speedup 1.0000x reference)
import jax
import jax.numpy as jnp
from jax.experimental import pallas as pl


def kernel(x, W):
    raise NotImplementedError("write your pallas kernel here")


# speedup vs baseline: 1.2772x; 1.2772x over previous
import functools

import jax
import jax.numpy as jnp
from jax import lax
from jax.experimental import pallas as pl
from jax.experimental.pallas import tpu as pltpu
from jax.experimental.pallas import tpu_sc as plsc

_BETA = 0.25


def _tc_body(flat_ref, x2_ref, w_ref, w2_ref, idx_ref, loss_ref):
    b = pl.program_id(0)
    T, C = flat_ref.shape
    K = w_ref.shape[0]

    fl = flat_ref[...]
    w = w_ref[...]
    x2 = x2_ref[...]
    w2 = w2_ref[...]

    s2 = lax.dot_general(fl + fl, w, (((1,), (1,)), ((), ())),
                         preferred_element_type=jnp.float32)
    d2 = (x2 + w2) - s2

    m = jnp.min(d2, axis=1, keepdims=True)
    lanes = lax.broadcasted_iota(jnp.int32, (T, K), 1).astype(jnp.float32)
    cand = jnp.where(d2 == m, lanes, jnp.float32(K))
    idxf = jnp.min(cand, axis=1, keepdims=True)
    idx_ref[...] = idxf.astype(jnp.int32)

    part = jnp.sum(m, keepdims=True)
    @pl.when(b == 0)
    def _():
        loss_ref[...] = jnp.zeros((1, 1), jnp.float32)
    loss_ref[...] += part


def _sc_lookup(wt, idx, B, C, T, K):
    info = plsc.get_sparse_core_info()
    nc, ns, L = info.num_cores, info.num_subcores, info.num_lanes
    nw = nc * ns
    cpw = C // (nw // B)
    mesh = plsc.VectorSubcoreMesh(core_axis_name="c", subcore_axis_name="s")

    @functools.partial(
        pl.kernel, mesh=mesh,
        out_type=jax.ShapeDtypeStruct((B, C, T), jnp.float32),
        compiler_params=pltpu.CompilerParams(needs_layout_passes=False),
        scratch_types=[
            pltpu.VMEM((cpw * K,), jnp.float32),
            pltpu.VMEM((T,), jnp.int32),
            pltpu.VMEM((cpw, T), jnp.float32),
        ],
    )
    def body(wt_hbm, idx_hbm, out_hbm, wt_v, idx_v, o_v):
        wid = lax.axis_index("s") * nc + lax.axis_index("c")
        b = wid // 2
        c0 = (wid % 2) * cpw
        pltpu.sync_copy(wt_hbm.at[pl.ds(c0 * K, cpw * K)], wt_v)
        pltpu.sync_copy(idx_hbm.at[b], idx_v)

        def chunk(ch, carry):
            iv = idx_v[pl.ds(ch * L, L)]
            for cc in range(cpw):
                g = plsc.load_gather(wt_v, [iv + (cc * K)])
                o_v[cc, pl.ds(ch * L, L)] = g
            return carry

        lax.fori_loop(0, T // L, chunk, 0)
        pltpu.sync_copy(o_v, out_hbm.at[b, pl.ds(c0, cpw)])

    return body(wt, idx)


@jax.jit
def kernel(x, W):
    B, C, T = x.shape
    K = W.shape[0]
    flat = jnp.transpose(x, (0, 2, 1)).reshape(B * T, C)
    x2 = jnp.sum(flat * flat, axis=1, keepdims=True)
    w2 = jnp.sum(W * W, axis=1)[None, :]

    idx2, loss_sum = pl.pallas_call(
        _tc_body,
        grid=(B,),
        in_specs=[
            pl.BlockSpec((T, C), lambda b: (b, 0)),
            pl.BlockSpec((T, 1), lambda b: (b, 0)),
            pl.BlockSpec((K, C), lambda b: (0, 0)),
            pl.BlockSpec((1, K), lambda b: (0, 0)),
        ],
        out_specs=[
            pl.BlockSpec((T, 1), lambda b: (b, 0)),
            pl.BlockSpec((1, 1), lambda b: (0, 0)),
        ],
        out_shape=[
            jax.ShapeDtypeStruct((B * T, 1), jnp.int32),
            jax.ShapeDtypeStruct((1, 1), jnp.float32),
        ],
    )(flat, x2, W, w2)

    idx = idx2.reshape(B, T)
    qout = _sc_lookup(W.T.reshape(-1), idx, B, C, T, K)

    codebook_loss = loss_sum[0, 0] / (B * C * T)
    commitment_loss = _BETA * codebook_loss
    return qout, codebook_loss, commitment_loss, idx
